# Initial kernel scaffold; baseline (speedup 1.0000x reference)
import jax
import jax.numpy as jnp
from jax.experimental import pallas as pl


def kernel(x, edge_index, edge_weight, W1, b1, W2, b2, Wfc, bfc):
    raise NotImplementedError("write your pallas kernel here")


# speedup vs baseline: 10.2960x; 10.2960x over previous
import functools

import jax
import jax.numpy as jnp
from jax import lax
from jax.experimental import pallas as pl
from jax.experimental.pallas import tpu as pltpu
from jax.experimental.pallas import tpu_sc as plsc

N = 10000
E = 320000
D = 128
NC = 2
NS = 16
NW = NC * NS
EPT = E // NW
K = 80
NCH = EPT // K
NP_ = 10240
RPT = NP_ // NS
ZR = 128

_mesh = plsc.VectorSubcoreMesh(
    core_axis_name="c", subcore_axis_name="s", num_cores=NC, num_subcores=NS)


def _deg_body(col_hbm, w_hbm, out_hbm, colv, wv, degl):
    c = lax.axis_index("c")
    s = lax.axis_index("s")
    wid = c * NS + s
    base = wid * EPT

    def zero(i, _):
        degl[pl.ds(i * 16, 16)] = jnp.zeros((16,), jnp.float32)
        return 0
    lax.fori_loop(0, NP_ // 16, zero, 0)

    pltpu.sync_copy(col_hbm.at[pl.ds(base, EPT)], colv)
    pltpu.sync_copy(w_hbm.at[pl.ds(base, EPT)], wv)

    def scat(i, _):
        c16 = colv[pl.ds(i * 16, 16)]
        w16 = wv[pl.ds(i * 16, 16)]
        plsc.addupdate_scatter(degl, [c16], w16)
        return 0
    lax.fori_loop(0, EPT // 16, scat, 0)

    pltpu.sync_copy(degl, out_hbm.at[wid, 0])


_sc_params = pltpu.CompilerParams(needs_layout_passes=False)

_deg_call = pl.kernel(
    _deg_body,
    out_type=jax.ShapeDtypeStruct((NW, 1, NP_), jnp.float32),
    mesh=_mesh,
    compiler_params=_sc_params,
    scratch_types=[
        pltpu.VMEM((EPT,), jnp.int32),
        pltpu.VMEM((EPT,), jnp.float32),
        pltpu.VMEM((NP_,), jnp.float32),
    ],
)


def _agg_body(g_hbm, row_hbm, col_hbm, w_hbm, out_hbm,
              idxrow, idxcol, wbuf, rows, zbuf, accs, gsem):
    c = lax.axis_index("c")
    s = lax.axis_index("s")
    base = (c * NS + s) * EPT

    def zrow(i, _):
        for q in range(8):
            zbuf[i, pl.ds(q * 16, 16)] = jnp.zeros((16,), jnp.float32)
        return 0
    lax.fori_loop(0, ZR, zrow, 0)
    for k in range(RPT // ZR):
        pltpu.sync_copy(zbuf, accs.at[pl.ds(s * RPT + k * ZR, ZR)])
    plsc.subcore_barrier()

    def chunk(j, _):
        off = base + j * K
        pltpu.sync_copy(row_hbm.at[pl.ds(off, K)], idxrow)
        pltpu.sync_copy(col_hbm.at[pl.ds(off, K)], idxcol)
        pltpu.sync_copy(w_hbm.at[pl.ds(off, K)], wbuf)
        pltpu.async_copy(g_hbm.at[idxrow], rows, gsem).wait()

        def scale(e, _):
            wspl = plsc.load_gather(wbuf, [jnp.full((16,), e, jnp.int32)])
            for q in range(8):
                sl = pl.ds(q * 16, 16)
                rows[e, sl] = rows[e, sl] * wspl
            return 0
        lax.fori_loop(0, K, scale, 0)

        pltpu.sync_copy(rows, accs.at[idxcol], add=True)
        return 0
    lax.fori_loop(0, NCH, chunk, 0)

    plsc.subcore_barrier()
    for k in range(RPT // ZR):
        off = s * RPT + k * ZR
        pltpu.sync_copy(accs.at[pl.ds(off, ZR)], out_hbm.at[c, pl.ds(off, ZR)])


_agg_call = pl.kernel(
    _agg_body,
    out_type=jax.ShapeDtypeStruct((NC, NP_, D), jnp.float32),
    mesh=_mesh,
    compiler_params=_sc_params,
    scratch_types=[
        pltpu.VMEM((K,), jnp.int32),
        pltpu.VMEM((K,), jnp.int32),
        pltpu.VMEM((K,), jnp.float32),
        pltpu.VMEM((K, D), jnp.float32),
        pltpu.VMEM((ZR, D), jnp.float32),
        pltpu.VMEM_SHARED((NP_, D), jnp.float32),
        pltpu.SemaphoreType.DMA,
    ],
)

BM = 2048
_GRID = NP_ // BM


def _dis_of(degp_ref):
    i = pl.program_id(0)
    deg = jnp.sum(degp_ref[:, 0, pl.ds(i * BM, BM)], axis=0) + 1.0
    return jnp.where(deg > 0, lax.rsqrt(jnp.maximum(deg, 1e-12)), 0.0)


def _tc1_body(x_ref, w_ref, degp_ref, o_ref):
    dis = _dis_of(degp_ref)
    h = jnp.dot(x_ref[...], w_ref[...], preferred_element_type=jnp.float32)
    o_ref[...] = dis[:, None] * h


def _tc2_body(p_ref, g_ref, degp_ref, b_ref, w_ref, o_ref):
    dis = _dis_of(degp_ref)
    s = p_ref[0] + p_ref[1] + g_ref[...]
    out1 = dis[:, None] * s + b_ref[...]
    h = jnp.dot(out1, w_ref[...], preferred_element_type=jnp.float32)
    o_ref[...] = dis[:, None] * h


def _tc3_body(p_ref, g_ref, degp_ref, b_ref, w_ref, bfc_ref, o_ref):
    dis = _dis_of(degp_ref)
    s = p_ref[0] + p_ref[1] + g_ref[...]
    out2 = dis[:, None] * s + b_ref[...]
    y = jnp.dot(out2, w_ref[...], preferred_element_type=jnp.float32)
    y = y + bfc_ref[...]
    o_ref[...] = jnp.round(jnp.clip(y, 0.0, 10.0))


_tc1 = pl.pallas_call(
    _tc1_body,
    grid=(_GRID,),
    in_specs=[
        pl.BlockSpec((BM, D), lambda i: (i, 0)),
        pl.BlockSpec((D, D), lambda i: (0, 0)),
        pl.BlockSpec((NW, 1, NP_), lambda i: (0, 0, 0)),
    ],
    out_specs=pl.BlockSpec((BM, D), lambda i: (i, 0)),
    out_shape=jax.ShapeDtypeStruct((NP_, D), jnp.float32),
)

_tc2 = pl.pallas_call(
    _tc2_body,
    grid=(_GRID,),
    in_specs=[
        pl.BlockSpec((NC, BM, D), lambda i: (0, i, 0)),
        pl.BlockSpec((BM, D), lambda i: (i, 0)),
        pl.BlockSpec((NW, 1, NP_), lambda i: (0, 0, 0)),
        pl.BlockSpec((1, D), lambda i: (0, 0)),
        pl.BlockSpec((D, D), lambda i: (0, 0)),
    ],
    out_specs=pl.BlockSpec((BM, D), lambda i: (i, 0)),
    out_shape=jax.ShapeDtypeStruct((NP_, D), jnp.float32),
)

_tc3 = pl.pallas_call(
    _tc3_body,
    grid=(_GRID,),
    in_specs=[
        pl.BlockSpec((NC, BM, D), lambda i: (0, i, 0)),
        pl.BlockSpec((BM, D), lambda i: (i, 0)),
        pl.BlockSpec((NW, 1, NP_), lambda i: (0, 0, 0)),
        pl.BlockSpec((1, D), lambda i: (0, 0)),
        pl.BlockSpec((D, 1), lambda i: (0, 0)),
        pl.BlockSpec((1, 1), lambda i: (0, 0)),
    ],
    out_specs=pl.BlockSpec((BM, 1), lambda i: (i, 0)),
    out_shape=jax.ShapeDtypeStruct((NP_, 1), jnp.float32),
)


def kernel(x, edge_index, edge_weight, W1, b1, W2, b2, Wfc, bfc):
    row = edge_index[0]
    col = edge_index[1]
    x_p = jnp.pad(x, ((0, NP_ - N), (0, 0)))
    degp = _deg_call(col, edge_weight)
    g1 = _tc1(x_p, W1, degp)
    p = _agg_call(g1, row, col, edge_weight)
    g2 = _tc2(p, g1, degp, b1.reshape(1, D), W2)
    q = _agg_call(g2, row, col, edge_weight)
    y = _tc3(q, g2, degp, b2.reshape(1, D), Wfc, bfc.reshape(1, 1))
    return y[:N]
